# baseline (speedup 1.0000x reference)
import functools

import jax
import jax.numpy as jnp
from jax import lax
from jax.experimental import pallas as pl
from jax.experimental.pallas import tpu as pltpu
from jax.experimental.pallas import tpu_sc as plsc

N = 10000
D = 128
H = 256
C = 527
K = 16
TW = 1.0

KP1 = K + 1
RB = 400
LB = 1000

NEG = -3.0e38
BIGI = 2**30


def _simtop_body(xb_ref, xnt_ref, vals_ref, idx_ref):
    xb = xb_ref[...]
    xnt = xnt_ref[...]
    sim = jax.lax.dot_general(
        xb, xnt, (((1,), (0,)), ((), ())),
        preferred_element_type=jnp.float32)
    col = jax.lax.broadcasted_iota(jnp.int32, (RB, N), 1)

    def step(t, carry):
        vprev, pprev, vals, idx = carry
        elig = (sim < vprev[:, None]) | (
            (sim == vprev[:, None]) & (col > pprev[:, None]))
        m = jnp.max(jnp.where(elig, sim, NEG), axis=1)
        pos = jnp.min(jnp.where(elig & (sim == m[:, None]), col, BIGI), axis=1)
        lane = jax.lax.broadcasted_iota(jnp.int32, (RB, KP1), 1)
        vals = jnp.where(lane == t, m[:, None], vals)
        idx = jnp.where(lane == t, pos[:, None], idx)
        return m, pos, vals, idx

    vals0 = jnp.full((RB, KP1), NEG, jnp.float32)
    idx0 = jnp.zeros((RB, KP1), jnp.int32)
    vp0 = jnp.full((RB,), jnp.inf, jnp.float32)
    pp0 = jnp.full((RB,), -1, jnp.int32)
    _, _, vals, idx = jax.lax.fori_loop(0, KP1, step, (vp0, pp0, vals0, idx0))
    vals_ref[...] = vals
    idx_ref[...] = idx


def _simtop(xn):
    xnt = xn.T
    return pl.pallas_call(
        _simtop_body,
        grid=(N // RB,),
        in_specs=[
            pl.BlockSpec((RB, D), lambda i: (i, 0)),
            pl.BlockSpec((D, N), lambda i: (0, 0)),
        ],
        out_specs=[
            pl.BlockSpec((RB, KP1), lambda i: (i, 0)),
            pl.BlockSpec((RB, KP1), lambda i: (i, 0)),
        ],
        out_shape=[
            jax.ShapeDtypeStruct((N, KP1), jnp.float32),
            jax.ShapeDtypeStruct((N, KP1), jnp.int32),
        ],
    )(xn, xnt)


E18 = 18
NE = N * E18
NT = 16
ROWS = N // NT
CH = 128
EB = 13312
RSTRIDE = 624


def _make_segsum(F):
    Fh = F // 2
    nv = Fh // 16
    mesh = plsc.VectorSubcoreMesh(core_axis_name="c", subcore_axis_name="s")

    @functools.partial(
        pl.kernel,
        out_type=jax.ShapeDtypeStruct((2, N, Fh), jnp.float32),
        mesh=mesh,
        scratch_types=[
            pltpu.VMEM((CH,), jnp.int32),
            pltpu.VMEM((CH,), jnp.int32),
            pltpu.VMEM((CH, 16), jnp.float32),
            pltpu.VMEM((CH, Fh), jnp.float32),
            pltpu.VMEM((CH, Fh), jnp.float32),
            pltpu.VMEM_SHARED((N, Fh), jnp.float32),
            pltpu.SemaphoreType.DMA,
            pltpu.SemaphoreType.DMA,
        ],
    )
    def seg_kernel(xstack, dstP, srcP, wP16, out, cidx, cdl, w16, gbuf, msgs,
                   acc, sem, semm):
        c = lax.axis_index("c")
        s = lax.axis_index("s")
        coff = c * N
        z16 = jnp.zeros((16,), jnp.float32)

        for e in range(CH):
            for v in range(nv):
                msgs[e, pl.ds(16 * v, 16)] = z16
        r0 = s * RSTRIDE
        for o in range(5):
            pltpu.sync_copy(msgs.at[pl.ds(0, CH)], acc.at[pl.ds(r0 + o * CH, CH)])
        plsc.subcore_barrier()

        def chunk(ch, _):
            cb = ch * CH
            pltpu.sync_copy(srcP.at[s, pl.ds(cb, CH)], cidx)
            for u in range(CH // 16):
                cidx[pl.ds(16 * u, 16)] = cidx[pl.ds(16 * u, 16)] + coff
            cp = pltpu.async_copy(xstack.at[cidx], gbuf, sem)
            cd_ = pltpu.async_copy(dstP.at[s, ch], cdl, semm)
            cw_ = pltpu.async_copy(wP16.at[s, pl.ds(cb, CH)], w16, semm)
            cd_.wait()
            cw_.wait()
            cp.wait()

            def edge(eb, _):
                for j in range(4):
                    e = eb * 4 + j
                    wv16 = w16[e, pl.ds(0, 16)]
                    for v in range(nv):
                        msgs[e, pl.ds(16 * v, 16)] = (
                            gbuf[e, pl.ds(16 * v, 16)] * wv16)
                return 0
            lax.fori_loop(0, CH // 4, edge, 0)
            pltpu.sync_copy(msgs, acc.at[cdl], add=True)
            return 0
        lax.fori_loop(0, EB // CH, chunk, 0)
        plsc.subcore_barrier()

        pltpu.sync_copy(acc.at[pl.ds(r0, 5 * CH)], out.at[c, pl.ds(r0, 5 * CH)])

    return seg_kernel


def _segsum_sc(x, dstP, srcP, wP16):
    F = x.shape[1]
    if F < H:
        xp = jnp.concatenate([x, jnp.zeros((N, H - F), jnp.float32)], axis=1)
        return _segsum_sc(xp, dstP, srcP, wP16)[:, :F]
    Fh = F // 2
    xstack = jnp.concatenate([x[:, :Fh], x[:, Fh:]], axis=0)
    out = _make_segsum(F)(xstack, dstP, srcP, wP16)
    return jnp.concatenate([out[0], out[1]], axis=1)


def _ln(h, g, b):
    m = jnp.mean(h, axis=-1, keepdims=True)
    v = jnp.mean((h - m) ** 2, axis=-1, keepdims=True)
    return (h - m) / jnp.sqrt(v + 1e-5) * g + b


def _layer1_body(agg_ref, x_ref, wrel_ref, brel_ref, wroot_ref, resw_ref,
                 resb_ref, g_ref, b_ref, h_ref):
    x = x_ref[...]
    h = (jnp.dot(agg_ref[...], wrel_ref[...], preferred_element_type=jnp.float32)
         + brel_ref[...]
         + jnp.dot(x, wroot_ref[...], preferred_element_type=jnp.float32))
    r = jnp.dot(x, resw_ref[...], preferred_element_type=jnp.float32) + resb_ref[...]
    h_ref[...] = _ln(jax.nn.relu(h) + r, g_ref[...], b_ref[...])


def _layer_body(agg_ref, x_ref, wrel_ref, brel_ref, wroot_ref, g_ref, b_ref,
                h_ref):
    x = x_ref[...]
    h = (jnp.dot(agg_ref[...], wrel_ref[...], preferred_element_type=jnp.float32)
         + brel_ref[...]
         + jnp.dot(x, wroot_ref[...], preferred_element_type=jnp.float32))
    h_ref[...] = _ln(jax.nn.relu(h) + x, g_ref[...], b_ref[...])


def _layer1(agg, x, W_rel, b_rel, W_root, res_W, res_b, g, b):
    fin = x.shape[1]
    return pl.pallas_call(
        _layer1_body,
        grid=(N // LB,),
        in_specs=[
            pl.BlockSpec((LB, fin), lambda i: (i, 0)),
            pl.BlockSpec((LB, fin), lambda i: (i, 0)),
            pl.BlockSpec((fin, H), lambda i: (0, 0)),
            pl.BlockSpec((H,), lambda i: (0,)),
            pl.BlockSpec((fin, H), lambda i: (0, 0)),
            pl.BlockSpec((fin, H), lambda i: (0, 0)),
            pl.BlockSpec((H,), lambda i: (0,)),
            pl.BlockSpec((H,), lambda i: (0,)),
            pl.BlockSpec((H,), lambda i: (0,)),
        ],
        out_specs=pl.BlockSpec((LB, H), lambda i: (i, 0)),
        out_shape=jax.ShapeDtypeStruct((N, H), jnp.float32),
    )(agg, x, W_rel, b_rel, W_root, res_W, res_b, g, b)


def _layer(agg, x, W_rel, b_rel, W_root, g, b):
    return pl.pallas_call(
        _layer_body,
        grid=(N // LB,),
        in_specs=[
            pl.BlockSpec((LB, H), lambda i: (i, 0)),
            pl.BlockSpec((LB, H), lambda i: (i, 0)),
            pl.BlockSpec((H, H), lambda i: (0, 0)),
            pl.BlockSpec((H,), lambda i: (0,)),
            pl.BlockSpec((H, H), lambda i: (0, 0)),
            pl.BlockSpec((H,), lambda i: (0,)),
            pl.BlockSpec((H,), lambda i: (0,)),
        ],
        out_specs=pl.BlockSpec((LB, H), lambda i: (i, 0)),
        out_shape=jax.ShapeDtypeStruct((N, H), jnp.float32),
    )(agg, x, W_rel, b_rel, W_root, g, b)


def _fc_body(h_ref, w_ref, b_ref, o_ref):
    o_ref[...] = (jnp.dot(h_ref[...], w_ref[...],
                          preferred_element_type=jnp.float32) + b_ref[...])


def _fc(h, fc_W, fc_b):
    return pl.pallas_call(
        _fc_body,
        grid=(N // LB,),
        in_specs=[
            pl.BlockSpec((LB, H), lambda i: (i, 0)),
            pl.BlockSpec((H, C), lambda i: (0, 0)),
            pl.BlockSpec((C,), lambda i: (0,)),
        ],
        out_specs=pl.BlockSpec((LB, C), lambda i: (i, 0)),
        out_shape=jax.ShapeDtypeStruct((N, C), jnp.float32),
    )(h, fc_W, fc_b)


def kernel(x, bn_gamma, bn_beta, bn_mean, bn_var, res_W, res_b, W_rel1, b_rel1,
           W_root1, W_rel2, b_rel2, W_root2, W_rel3, b_rel3, W_root3,
           ln1_g, ln1_b, ln2_g, ln2_b, ln3_g, ln3_b, fc_W, fc_b):
    x = (x - bn_mean) / jnp.sqrt(bn_var + 1e-5) * bn_gamma + bn_beta
    xs = jax.lax.stop_gradient(x)
    xn = xs / (jnp.linalg.norm(xs, axis=1, keepdims=True) + 1e-8)

    vals, idx = _simtop(xn)
    nbrs = idx[:, 1:]
    v = vals[:, 1:]
    rng = jnp.arange(N, dtype=jnp.int32)
    w_knn = v + TW * (jnp.abs(nbrs - rng[:, None]) == 1).astype(jnp.float32)

    present = jnp.any(nbrs[:-1] == (rng[:-1] + 1)[:, None], axis=1)
    wt = jnp.where(present, 0.0, TW).astype(jnp.float32)
    w_fwd = jnp.concatenate([wt, jnp.zeros((1,), jnp.float32)])
    w_bwd = jnp.concatenate([jnp.zeros((1,), jnp.float32), wt])
    d_fwd = jnp.minimum(rng + 1, N - 1)
    d_bwd = jnp.maximum(rng - 1, 0)
    dstT = jnp.concatenate([nbrs, d_fwd[:, None], d_bwd[:, None]], axis=1)
    wT = jnp.concatenate([w_knn, w_fwd[:, None], w_bwd[:, None]], axis=1)
    srcT = jnp.broadcast_to(rng[:, None], (N, E18))

    dstE = dstT.reshape(-1)
    srcE = srcT.reshape(-1)
    wE = wT.reshape(-1)
    owner = jnp.minimum(dstE // RSTRIDE, NT - 1)
    order = jnp.argsort(owner, stable=True)
    ownerS = owner[order]
    counts = jnp.bincount(owner, length=NT).astype(jnp.int32)
    off = jnp.concatenate([jnp.zeros((1,), jnp.int32),
                           jnp.cumsum(counts)[:-1].astype(jnp.int32)])
    rank = jnp.arange(NE, dtype=jnp.int32) - off[ownerS]
    flatpos = jnp.where(rank < EB, ownerS * EB + rank, jnp.int32(NT * EB))
    dstP = jnp.zeros((NT * EB,), jnp.int32).at[flatpos].set(
        dstE[order], mode="drop").reshape(NT, EB // CH, CH)
    srcP = jnp.zeros((NT * EB,), jnp.int32).at[flatpos].set(
        srcE[order], mode="drop").reshape(NT, EB)
    wP = jnp.zeros((NT * EB,), jnp.float32).at[flatpos].set(
        wE[order], mode="drop")
    wP16 = jnp.broadcast_to(wP.reshape(NT, EB, 1), (NT, EB, 16))

    agg1 = _segsum_sc(x, dstP, srcP, wP16)
    h = _layer1(agg1, x, W_rel1, b_rel1, W_root1, res_W, res_b, ln1_g, ln1_b)
    agg2 = _segsum_sc(h, dstP, srcP, wP16)
    h = _layer(agg2, h, W_rel2, b_rel2, W_root2, ln2_g, ln2_b)
    agg3 = _segsum_sc(h, dstP, srcP, wP16)
    h = _layer(agg3, h, W_rel3, b_rel3, W_root3, ln3_g, ln3_b)
    return _fc(h, fc_W, fc_b)
